# baseline (speedup 1.0000x reference)
import jax
import jax.numpy as jnp
from jax import lax
from jax.experimental import pallas as pl
from jax.experimental.pallas import tpu as pltpu
from jax.experimental.pallas import tpu_sc as plsc

VOCAB = 1000000
ROWS = 65536
N_CHUNKS = 8
CHUNK = 16
B = 16384

NC = 2
NS = 16
NW = NC * NS
TPW = B // NW
RPW = TPW * N_CHUNKS
SUB = 4
CPS = N_CHUNKS // SUB
RPS = TPW * CPS
IDX_W = 128
KPW = RPW // IDX_W
K_SUB = RPS // IDX_W


def _body(x_hbm, t0_hbm, t1_hbm, w_hbm, h0_hbm, h1_hbm, h2_hbm, out_hbm,
          x_v, xe_v, g0_v, g1_v, g2_v, w0_v, w1_v, p0_v, p1_v, out_v, sem):
    c = lax.axis_index("c")
    s = lax.axis_index("s")
    wid = s * NC + c
    tok_base = wid * TPW

    pltpu.sync_copy(x_hbm.at[pl.ds(tok_base, TPW)], x_v)

    def xe_body(i, _):
        cc = lax.shift_right_logical(i, 5)
        bo = jnp.bitwise_and(i, 31) * 16
        xe_v[pl.ds(i * 16, 16)] = x_v[pl.ds(bo, 16)] * N_CHUNKS + cc
        return 0

    lax.fori_loop(0, RPW // 16, xe_body, 0)

    cps = []
    for kk in range(KPW):
        sl = pl.ds(kk * IDX_W, IDX_W)
        cps.append(pltpu.async_copy(h0_hbm.at[xe_v.at[sl]], g0_v.at[sl], sem))
        cps.append(pltpu.async_copy(h1_hbm.at[xe_v.at[sl]], g1_v.at[sl], sem))
        cps.append(pltpu.async_copy(h2_hbm.at[xe_v.at[sl]], g2_v.at[sl], sem))
    for cp in cps:
        cp.wait()

    def idx_body(i, _):
        cc = lax.shift_right_logical(i, 5)
        sl = pl.ds(i * 16, 16)
        g0_v[sl] = g0_v[sl] * N_CHUNKS + cc
        g1_v[sl] = g1_v[sl] * N_CHUNKS + cc
        i2 = g2_v[sl] * N_CHUNKS + cc
        xe_v[sl] = i2 * 2
        g2_v[sl] = i2 * 2 + 1
        return 0

    lax.fori_loop(0, RPW // 16, idx_body, 0)

    cps = []
    for kk in range(KPW):
        sl = pl.ds(kk * IDX_W, IDX_W)
        cps.append(pltpu.async_copy(w_hbm.at[xe_v.at[sl]], w0_v.at[sl], sem))
        cps.append(pltpu.async_copy(w_hbm.at[g2_v.at[sl]], w1_v.at[sl], sem))
    for cp in cps:
        cp.wait()

    for sb in range(SUB):
        cps = []
        for j in range(K_SUB):
            sl = pl.ds((sb * K_SUB + j) * IDX_W, IDX_W)
            dst = pl.ds(j * IDX_W, IDX_W)
            cps.append(pltpu.async_copy(t0_hbm.at[g0_v.at[sl]],
                                        p0_v.at[dst], sem))
            cps.append(pltpu.async_copy(t1_hbm.at[g1_v.at[sl]],
                                        p1_v.at[dst], sem))
        for cp in cps:
            cp.wait()

        def row_body(m, _):
            q = sb * RPS + m
            qv = jnp.full((16,), q, jnp.int32)
            w0 = plsc.load_gather(w0_v, [qv])
            w1 = plsc.load_gather(w1_v, [qv])
            cc = sb * CPS + lax.shift_right_logical(m, 9)
            b = jnp.bitwise_and(m, TPW - 1)
            out_v[b, pl.ds(cc * CHUNK, CHUNK)] = (
                p0_v[m, :] * w0 + p1_v[m, :] * w1)
            return 0

        lax.fori_loop(0, RPS, row_body, 0)

    pltpu.sync_copy(out_v, out_hbm.at[pl.ds(tok_base, TPW), :])


@jax.jit
def _call(x, t0, t1, w, h0f, h1f, h2f):
    mesh = plsc.VectorSubcoreMesh(core_axis_name="c", subcore_axis_name="s")
    run = pl.kernel(
        _body,
        out_type=jax.ShapeDtypeStruct((B, N_CHUNKS * CHUNK), jnp.float32),
        mesh=mesh,
        compiler_params=pltpu.CompilerParams(use_tc_tiling_on_sc=False,
                                             needs_layout_passes=False),
        scratch_types=[
            pltpu.VMEM((TPW,), jnp.int32),
            pltpu.VMEM((RPW,), jnp.int32),
            pltpu.VMEM((RPW,), jnp.int32),
            pltpu.VMEM((RPW,), jnp.int32),
            pltpu.VMEM((RPW,), jnp.int32),
            pltpu.VMEM((RPW,), jnp.float32),
            pltpu.VMEM((RPW,), jnp.float32),
            pltpu.VMEM((RPS, CHUNK), jnp.float32),
            pltpu.VMEM((RPS, CHUNK), jnp.float32),
            pltpu.VMEM((TPW, 128), jnp.float32),
            pltpu.SemaphoreType.DMA,
        ],
    )
    return run(x, t0, t1, w, h0f, h1f, h2f)


def kernel(x, table0, table1, weights, h0, h1, h2):
    t0 = table0.reshape(ROWS * N_CHUNKS, CHUNK)
    t1 = table1.reshape(ROWS * N_CHUNKS, CHUNK)
    w = weights.reshape(ROWS * N_CHUNKS * 2)
    h0f = h0.reshape(VOCAB * N_CHUNKS)
    h1f = h1.reshape(VOCAB * N_CHUNKS)
    h2f = h2.reshape(VOCAB * N_CHUNKS)
    return _call(x, t0, t1, w, h0f, h1f, h2f)


# speedup vs baseline: 1.0007x; 1.0007x over previous
import jax
import jax.numpy as jnp
from jax import lax
from jax.experimental import pallas as pl
from jax.experimental.pallas import tpu as pltpu
from jax.experimental.pallas import tpu_sc as plsc

VOCAB = 1000000
ROWS = 65536
N_CHUNKS = 8
CHUNK = 16
B = 16384

NC = 2
NS = 16
NW = NC * NS
TPW = B // NW
RPW = TPW * N_CHUNKS
SUB = 4
CPS = N_CHUNKS // SUB
RPS = TPW * CPS
IDX_W = 128
KPW = RPW // IDX_W
K_SUB = RPS // IDX_W


def _body(x_hbm, t0_hbm, t1_hbm, w_hbm, h0_hbm, h1_hbm, h2_hbm, out_hbm,
          x_v, xe_v, g0_v, g1_v, g2_v, w0_v, w1_v, p0_v, p1_v, out_v, sem):
    c = lax.axis_index("c")
    s = lax.axis_index("s")
    wid = s * NC + c
    tok_base = wid * TPW

    pltpu.sync_copy(x_hbm.at[pl.ds(tok_base, TPW)], x_v)

    def xe_body(i, _):
        cc = lax.shift_right_logical(i, 5)
        bo = jnp.bitwise_and(i, 31) * 16
        xe_v[pl.ds(i * 16, 16)] = x_v[pl.ds(bo, 16)] * N_CHUNKS + cc
        return 0

    lax.fori_loop(0, RPW // 16, xe_body, 0)

    def hfire(kk, _):
        sl = pl.ds(kk * IDX_W, IDX_W)
        pltpu.async_copy(h0_hbm.at[xe_v.at[sl]], g0_v.at[sl], sem)
        pltpu.async_copy(h1_hbm.at[xe_v.at[sl]], g1_v.at[sl], sem)
        pltpu.async_copy(h2_hbm.at[xe_v.at[sl]], g2_v.at[sl], sem)
        return 0

    lax.fori_loop(0, KPW, hfire, 0)
    for g in (g0_v, g1_v, g2_v):
        pltpu.make_async_copy(h0_hbm.at[pl.ds(0, RPW)], g, sem).wait()

    def idx_body(i, _):
        cc = lax.shift_right_logical(i, 5)
        sl = pl.ds(i * 16, 16)
        g0_v[sl] = g0_v[sl] * N_CHUNKS + cc
        g1_v[sl] = g1_v[sl] * N_CHUNKS + cc
        i2 = g2_v[sl] * N_CHUNKS + cc
        xe_v[sl] = i2 * 2
        g2_v[sl] = i2 * 2 + 1
        return 0

    lax.fori_loop(0, RPW // 16, idx_body, 0)

    def wfire(kk, _):
        sl = pl.ds(kk * IDX_W, IDX_W)
        pltpu.async_copy(w_hbm.at[xe_v.at[sl]], w0_v.at[sl], sem)
        pltpu.async_copy(w_hbm.at[g2_v.at[sl]], w1_v.at[sl], sem)
        return 0

    lax.fori_loop(0, KPW, wfire, 0)
    for wv in (w0_v, w1_v):
        pltpu.make_async_copy(w_hbm.at[pl.ds(0, RPW)], wv, sem).wait()

    for sb in range(SUB):
        def tfire(j, _):
            sl = pl.ds((sb * K_SUB + j) * IDX_W, IDX_W)
            dst = pl.ds(j * IDX_W, IDX_W)
            pltpu.async_copy(t0_hbm.at[g0_v.at[sl]], p0_v.at[dst], sem)
            pltpu.async_copy(t1_hbm.at[g1_v.at[sl]], p1_v.at[dst], sem)
            return 0

        lax.fori_loop(0, K_SUB, tfire, 0)
        pltpu.make_async_copy(t0_hbm.at[pl.ds(0, RPS), :], p0_v, sem).wait()
        pltpu.make_async_copy(t1_hbm.at[pl.ds(0, RPS), :], p1_v, sem).wait()

        def row_body(m, _):
            q = sb * RPS + m
            qv = jnp.full((16,), q, jnp.int32)
            w0 = plsc.load_gather(w0_v, [qv])
            w1 = plsc.load_gather(w1_v, [qv])
            cc = sb * CPS + lax.shift_right_logical(m, 9)
            b = jnp.bitwise_and(m, TPW - 1)
            out_v[b, pl.ds(cc * CHUNK, CHUNK)] = (
                p0_v[m, :] * w0 + p1_v[m, :] * w1)
            return 0

        lax.fori_loop(0, RPS, row_body, 0)

    pltpu.sync_copy(out_v, out_hbm.at[pl.ds(tok_base, TPW), :])


@jax.jit
def _call(x, t0, t1, w, h0f, h1f, h2f):
    mesh = plsc.VectorSubcoreMesh(core_axis_name="c", subcore_axis_name="s")
    run = pl.kernel(
        _body,
        out_type=jax.ShapeDtypeStruct((B, N_CHUNKS * CHUNK), jnp.float32),
        mesh=mesh,
        compiler_params=pltpu.CompilerParams(use_tc_tiling_on_sc=False,
                                             needs_layout_passes=False),
        scratch_types=[
            pltpu.VMEM((TPW,), jnp.int32),
            pltpu.VMEM((RPW,), jnp.int32),
            pltpu.VMEM((RPW,), jnp.int32),
            pltpu.VMEM((RPW,), jnp.int32),
            pltpu.VMEM((RPW,), jnp.int32),
            pltpu.VMEM((RPW,), jnp.float32),
            pltpu.VMEM((RPW,), jnp.float32),
            pltpu.VMEM((RPS, CHUNK), jnp.float32),
            pltpu.VMEM((RPS, CHUNK), jnp.float32),
            pltpu.VMEM((TPW, 128), jnp.float32),
            pltpu.SemaphoreType.DMA,
        ],
    )
    return run(x, t0, t1, w, h0f, h1f, h2f)


def kernel(x, table0, table1, weights, h0, h1, h2):
    t0 = table0.reshape(ROWS * N_CHUNKS, CHUNK)
    t1 = table1.reshape(ROWS * N_CHUNKS, CHUNK)
    w = weights.reshape(ROWS * N_CHUNKS * 2)
    h0f = h0.reshape(VOCAB * N_CHUNKS)
    h1f = h1.reshape(VOCAB * N_CHUNKS)
    h2f = h2.reshape(VOCAB * N_CHUNKS)
    return _call(x, t0, t1, w, h0f, h1f, h2f)


# speedup vs baseline: 1.0007x; 1.0001x over previous
import jax
import jax.numpy as jnp
from jax import lax
from jax.experimental import pallas as pl
from jax.experimental.pallas import tpu as pltpu
from jax.experimental.pallas import tpu_sc as plsc

VOCAB = 1000000
ROWS = 65536
N_CHUNKS = 8
CHUNK = 16
B = 16384

NC = 2
NS = 16
NW = NC * NS
TPW = B // NW
RPW = TPW * N_CHUNKS
SUB = 4
CPS = N_CHUNKS // SUB
RPS = TPW * CPS
IDX_W = 128
KPW = RPW // IDX_W
K_SUB = RPS // IDX_W


def _body(x_hbm, t0_hbm, t1_hbm, w_hbm, h0_hbm, h1_hbm, h2_hbm, out_hbm,
          x_v, xe_v, g0_v, g1_v, g2_v, w0_v, w1_v, p0_v, p1_v, out_v, sem):
    c = lax.axis_index("c")
    s = lax.axis_index("s")
    wid = s * NC + c
    tok_base = wid * TPW

    pltpu.sync_copy(x_hbm.at[pl.ds(tok_base, TPW)], x_v)

    def xe_body(i, _):
        cc = lax.shift_right_logical(i, 5)
        bo = jnp.bitwise_and(i, 31) * 16
        xe_v[pl.ds(i * 16, 16)] = x_v[pl.ds(bo, 16)] * N_CHUNKS + cc
        return 0

    lax.fori_loop(0, RPW // 16, xe_body, 0)

    cp0 = pltpu.async_copy(h0_hbm.at[xe_v], g0_v, sem)
    cp1 = pltpu.async_copy(h1_hbm.at[xe_v], g1_v, sem)
    cp2 = pltpu.async_copy(h2_hbm.at[xe_v], g2_v, sem)
    cp0.wait()
    cp1.wait()
    cp2.wait()

    def idx_body(i, _):
        cc = lax.shift_right_logical(i, 5)
        sl = pl.ds(i * 16, 16)
        g0_v[sl] = g0_v[sl] * N_CHUNKS + cc
        g1_v[sl] = g1_v[sl] * N_CHUNKS + cc
        i2 = g2_v[sl] * N_CHUNKS + cc
        xe_v[sl] = i2 * 2
        g2_v[sl] = i2 * 2 + 1
        return 0

    lax.fori_loop(0, RPW // 16, idx_body, 0)

    cpw0 = pltpu.async_copy(w_hbm.at[xe_v], w0_v, sem)
    cpw1 = pltpu.async_copy(w_hbm.at[g2_v], w1_v, sem)
    cpw0.wait()
    cpw1.wait()

    for sb in range(SUB):
        sl = pl.ds(sb * RPS, RPS)
        cpt0 = pltpu.async_copy(t0_hbm.at[g0_v.at[sl]], p0_v, sem)
        cpt1 = pltpu.async_copy(t1_hbm.at[g1_v.at[sl]], p1_v, sem)
        cpt0.wait()
        cpt1.wait()

        def row_body(m, _):
            q = sb * RPS + m
            qv = jnp.full((16,), q, jnp.int32)
            w0 = plsc.load_gather(w0_v, [qv])
            w1 = plsc.load_gather(w1_v, [qv])
            cc = sb * CPS + lax.shift_right_logical(m, 9)
            b = jnp.bitwise_and(m, TPW - 1)
            out_v[b, pl.ds(cc * CHUNK, CHUNK)] = (
                p0_v[m, :] * w0 + p1_v[m, :] * w1)
            return 0

        lax.fori_loop(0, RPS, row_body, 0)

    pltpu.sync_copy(out_v, out_hbm.at[pl.ds(tok_base, TPW), :])


@jax.jit
def _call(x, t0, t1, w, h0f, h1f, h2f):
    mesh = plsc.VectorSubcoreMesh(core_axis_name="c", subcore_axis_name="s")
    run = pl.kernel(
        _body,
        out_type=jax.ShapeDtypeStruct((B, N_CHUNKS * CHUNK), jnp.float32),
        mesh=mesh,
        compiler_params=pltpu.CompilerParams(use_tc_tiling_on_sc=False,
                                             needs_layout_passes=False),
        scratch_types=[
            pltpu.VMEM((TPW,), jnp.int32),
            pltpu.VMEM((RPW,), jnp.int32),
            pltpu.VMEM((RPW,), jnp.int32),
            pltpu.VMEM((RPW,), jnp.int32),
            pltpu.VMEM((RPW,), jnp.int32),
            pltpu.VMEM((RPW,), jnp.float32),
            pltpu.VMEM((RPW,), jnp.float32),
            pltpu.VMEM((RPS, CHUNK), jnp.float32),
            pltpu.VMEM((RPS, CHUNK), jnp.float32),
            pltpu.VMEM((TPW, 128), jnp.float32),
            pltpu.SemaphoreType.DMA,
        ],
    )
    return run(x, t0, t1, w, h0f, h1f, h2f)


def kernel(x, table0, table1, weights, h0, h1, h2):
    t0 = table0.reshape(ROWS * N_CHUNKS, CHUNK)
    t1 = table1.reshape(ROWS * N_CHUNKS, CHUNK)
    w = weights.reshape(ROWS * N_CHUNKS * 2)
    h0f = h0.reshape(VOCAB * N_CHUNKS)
    h1f = h1.reshape(VOCAB * N_CHUNKS)
    h2f = h2.reshape(VOCAB * N_CHUNKS)
    return _call(x, t0, t1, w, h0f, h1f, h2f)
